# baseline (speedup 1.0000x reference)
import functools

import jax
import jax.numpy as jnp
from jax import lax
from jax.experimental import pallas as pl
from jax.experimental.pallas import tpu as pltpu
from jax.experimental.pallas import tpu_sc as plsc

N = 50000
E = 800000
D = 64
H = 4
HD = 16
G = 512
EB = 128
NEB = E // EB
NC, NS = 2, 16
NW = NC * NS
NROW = N // NS
BE = 2000
BN = 2000
F32 = jnp.float32

_SC_PARAMS = dict(
    compiler_params=pltpu.CompilerParams(use_tc_tiling_on_sc=False),
)


def _mesh():
    return plsc.VectorSubcoreMesh(core_axis_name="c", subcore_axis_name="s")


def _sc_gather(tables, idx2ds, SB):
    NT = len(tables)
    NI = len(idx2ds)
    dds = [int(t.shape[1]) for t in tables]
    NSB = NEB // SB
    SBR = SB * EB

    outs = [jax.ShapeDtypeStruct((E, dds[tt]), F32)
            for tt in range(NT) for _ in range(NI)]
    scratch = []
    for b in range(2):
        for ii in range(NI):
            scratch.append(pltpu.VMEM((SB, EB), jnp.int32))
    for b in range(2):
        for tt in range(NT):
            for ii in range(NI):
                scratch.append(pltpu.VMEM((SBR, dds[tt]), F32))
    scratch += [pltpu.SemaphoreType.DMA] * 4

    @functools.partial(
        pl.kernel, out_type=outs, mesh=_mesh(),
        scratch_types=scratch, **_SC_PARAMS)
    def k(*refs):
        tab_h = refs[:NT]
        idx_h = refs[NT:NT + NI]
        out_h = refs[NT + NI:NT + NI + NT * NI]
        p = NT + NI + NT * NI
        iv = [[refs[p + b * NI + ii] for ii in range(NI)] for b in range(2)]
        p += 2 * NI
        buf = [[refs[p + b * NT * NI + q] for q in range(NT * NI)]
               for b in range(2)]
        p += 2 * NT * NI
        semG = [refs[p], refs[p + 1]]
        semW = [refs[p + 2], refs[p + 3]]

        wid = lax.axis_index("s") * NC + lax.axis_index("c")
        T = (NSB - wid + NW - 1) // NW

        def wait_wb(b):
            for q in range(NT * NI):
                pltpu.make_async_copy(
                    buf[b][q], out_h[q].at[pl.ds(0, SBR)], semW[b]).wait()

        def body(i, carry):
            for b in range(2):
                t = 2 * i + b

                @pl.when(t < T)
                def _(b=b, t=t):
                    sb = wid + t * NW
                    off = pl.multiple_of(sb * SBR, EB)

                    @pl.when(t >= 2)
                    def _():
                        wait_wb(b)

                    for ii in range(NI):
                        pltpu.sync_copy(idx_h[ii].at[pl.ds(sb * SB, SB)],
                                        iv[b][ii])
                    descs = []
                    for tt in range(NT):
                        for ii in range(NI):
                            for j in range(SB):
                                descs.append(pltpu.async_copy(
                                    tab_h[tt].at[iv[b][ii].at[j]],
                                    buf[b][tt * NI + ii].at[pl.ds(j * EB, EB)],
                                    semG[b]))
                    for dsc in descs:
                        dsc.wait()
                    for q in range(NT * NI):
                        pltpu.async_copy(buf[b][q], out_h[q].at[pl.ds(off, SBR)],
                                         semW[b])
            return carry

        lax.fori_loop(0, (T + 1) // 2, body, 0)
        wait_wb(0)
        wait_wb(1)

    return k(*tables, *idx2ds)


def _sc_s1(expl, dst2d, z8):
    SB = 5
    SBR = SB * EB
    NSB = NEB // SB

    @functools.partial(
        pl.kernel, out_type=jax.ShapeDtypeStruct((E, 8), F32), mesh=_mesh(),
        scratch_types=[
            pltpu.VMEM((SB, EB), jnp.int32), pltpu.VMEM((SB, EB), jnp.int32),
            pltpu.VMEM((SBR, 8), F32), pltpu.VMEM((SBR, 8), F32),
            pltpu.VMEM((SB, EB), jnp.int32), pltpu.VMEM((SB, EB), jnp.int32),
            pltpu.VMEM((SBR, 8), F32), pltpu.VMEM((SBR, 8), F32),
            pltpu.VMEM_SHARED((N, 8), F32),
            pltpu.SemaphoreType.DMA, pltpu.SemaphoreType.DMA,
            pltpu.SemaphoreType.DMA, pltpu.SemaphoreType.DMA,
            pltpu.SemaphoreType.DMA, pltpu.SemaphoreType.DMA,
        ], **_SC_PARAMS)
    def k(ex_h, di_h, z_h, s1d_h, ivS0, ivS1, bufS0, bufS1, ivG0, ivG1,
          bufG0, bufG1, acc, sS0, sS1, sG0, sG1, sW0, sW1):
        ivS = [ivS0, ivS1]
        bufS = [bufS0, bufS1]
        ivG = [ivG0, ivG1]
        bufG = [bufG0, bufG1]
        semS = [sS0, sS1]
        semG = [sG0, sG1]
        semW = [sW0, sW1]
        cid = lax.axis_index("c")
        sid = lax.axis_index("s")
        wid = sid * NC + cid
        r0 = pl.multiple_of(sid * NROW, NROW)
        pltpu.sync_copy(z_h.at[pl.ds(r0, NROW)], acc.at[pl.ds(r0, NROW)])
        plsc.subcore_barrier()

        Ts = (NSB - sid + NS - 1) // NS

        def wait_sc(b):
            for j in range(SB):
                pltpu.make_async_copy(bufS[b].at[pl.ds(j * EB, EB)],
                                      acc.at[ivS[b].at[j]], semS[b]).wait()

        def s_body(i, carry):
            for b in range(2):
                t = 2 * i + b

                @pl.when(t < Ts)
                def _(b=b, t=t):
                    sb = sid + t * NS
                    off = pl.multiple_of(sb * SBR, EB)

                    @pl.when(t >= 2)
                    def _():
                        wait_sc(b)

                    pltpu.sync_copy(di_h.at[pl.ds(sb * SB, SB)], ivS[b])
                    pltpu.sync_copy(ex_h.at[pl.ds(off, SBR)], bufS[b])
                    for j in range(SB):
                        pltpu.async_copy(bufS[b].at[pl.ds(j * EB, EB)],
                                         acc.at[ivS[b].at[j]], semS[b],
                                         add=True)
            return carry

        lax.fori_loop(0, (Ts + 1) // 2, s_body, 0)
        wait_sc(0)
        wait_sc(1)
        plsc.subcore_barrier()

        Tg = (NSB - wid + NW - 1) // NW

        def wait_wb(b):
            pltpu.make_async_copy(bufG[b], s1d_h.at[pl.ds(0, SBR)],
                                  semW[b]).wait()

        def g_body(i, carry):
            for b in range(2):
                t = 2 * i + b

                @pl.when(t < Tg)
                def _(b=b, t=t):
                    sb = wid + t * NW
                    off = pl.multiple_of(sb * SBR, EB)

                    @pl.when(t >= 2)
                    def _():
                        wait_wb(b)

                    pltpu.sync_copy(di_h.at[pl.ds(sb * SB, SB)], ivG[b])
                    descs = []
                    for j in range(SB):
                        descs.append(pltpu.async_copy(
                            acc.at[ivG[b].at[j]],
                            bufG[b].at[pl.ds(j * EB, EB)], semG[b]))
                    for dsc in descs:
                        dsc.wait()
                    pltpu.async_copy(bufG[b], s1d_h.at[pl.ds(off, SBR)],
                                     semW[b])
            return carry

        lax.fori_loop(0, (Tg + 1) // 2, g_body, 0)
        wait_wb(0)
        wait_wb(1)

    return k(expl, dst2d, z8)


def _sc_scatter_msg(msgs, dst2d, z16):
    SB = 5
    SBR = SB * EB
    NSB = NEB // SB

    @functools.partial(
        pl.kernel, out_type=[jax.ShapeDtypeStruct((N, HD), F32)] * 4,
        mesh=_mesh(),
        scratch_types=[
            pltpu.VMEM((SB, EB), jnp.int32), pltpu.VMEM((SB, EB), jnp.int32),
            pltpu.VMEM((SBR, HD), F32), pltpu.VMEM((SBR, HD), F32),
            pltpu.VMEM_SHARED((N, HD), F32),
            pltpu.SemaphoreType.DMA, pltpu.SemaphoreType.DMA,
        ], **_SC_PARAMS)
    def k(m0_h, m1_h, m2_h, m3_h, di_h, z_h, o0_h, o1_h, o2_h, o3_h,
          iv0, iv1, buf0, buf1, acc, sS0, sS1):
        iv = [iv0, iv1]
        buf = [buf0, buf1]
        semS = [sS0, sS1]
        m_h = [m0_h, m1_h, m2_h, m3_h]
        o_h = [o0_h, o1_h, o2_h, o3_h]
        cid = lax.axis_index("c")
        sid = lax.axis_index("s")
        r0 = pl.multiple_of(sid * NROW, NROW)
        Ts = (NSB - sid + NS - 1) // NS

        def wait_sc(b):
            for j in range(SB):
                pltpu.make_async_copy(buf[b].at[pl.ds(j * EB, EB)],
                                      acc.at[iv[b].at[j]], semS[b]).wait()

        for q in range(2):
            pltpu.sync_copy(z_h.at[pl.ds(r0, NROW)], acc.at[pl.ds(r0, NROW)])
            plsc.subcore_barrier()

            def s_body(i, carry, q=q):
                for b in range(2):
                    t = 2 * i + b

                    @pl.when(t < Ts)
                    def _(b=b, t=t):
                        sb = sid + t * NS
                        off = pl.multiple_of(sb * SBR, EB)

                        @pl.when(t >= 2)
                        def _():
                            wait_sc(b)

                        pltpu.sync_copy(di_h.at[pl.ds(sb * SB, SB)], iv[b])

                        @pl.when(cid == 0)
                        def _():
                            pltpu.sync_copy(m_h[q].at[pl.ds(off, SBR)], buf[b])

                        @pl.when(cid == 1)
                        def _():
                            pltpu.sync_copy(m_h[2 + q].at[pl.ds(off, SBR)],
                                            buf[b])

                        for j in range(SB):
                            pltpu.async_copy(buf[b].at[pl.ds(j * EB, EB)],
                                             acc.at[iv[b].at[j]], semS[b],
                                             add=True)
                return carry

            lax.fori_loop(0, (Ts + 1) // 2, s_body, 0)
            wait_sc(0)
            wait_sc(1)
            plsc.subcore_barrier()

            @pl.when(cid == 0)
            def _(q=q):
                pltpu.sync_copy(acc.at[pl.ds(r0, NROW)],
                                o_h[q].at[pl.ds(r0, NROW)])

            @pl.when(cid == 1)
            def _(q=q):
                pltpu.sync_copy(acc.at[pl.ds(r0, NROW)],
                                o_h[2 + q].at[pl.ds(r0, NROW)])
            plsc.subcore_barrier()

    return k(*msgs, dst2d, z16)


def _tc_edge_l1(f, hs, hd, Wni, Wnj, Wfij, Wnode, attnM):
    fin = f.shape[1]

    def body(f_ref, hs_ref, hd_ref, wi_ref, wj_ref, wf_ref, wn_ref, am_ref,
             fo_ref, ex_ref, hn_ref):
        hs = hs_ref[...]
        fo = jnp.dot(hs, wi_ref[...], preferred_element_type=F32) \
            + jnp.dot(hd_ref[...], wj_ref[...], preferred_element_type=F32) \
            + jnp.dot(f_ref[...], wf_ref[...], preferred_element_type=F32)
        fo_ref[...] = fo
        lr = jnp.where(fo >= 0.0, fo, 0.2 * fo)
        l8 = jnp.dot(lr, am_ref[...], preferred_element_type=F32)
        ex_ref[...] = jnp.exp(jnp.clip(l8, -80.0, 80.0))
        hn_ref[...] = jnp.dot(hs, wn_ref[...], preferred_element_type=F32)

    c = lambda i: (i, 0)
    z = lambda i: (0, 0)
    return pl.pallas_call(
        body,
        grid=(E // BE,),
        in_specs=[
            pl.BlockSpec((BE, fin), c), pl.BlockSpec((BE, 32), c),
            pl.BlockSpec((BE, 32), c), pl.BlockSpec((32, D), z),
            pl.BlockSpec((32, D), z), pl.BlockSpec((fin, D), z),
            pl.BlockSpec((32, D), z), pl.BlockSpec((D, 8), z),
        ],
        out_specs=[pl.BlockSpec((BE, D), c), pl.BlockSpec((BE, 8), c),
                   pl.BlockSpec((BE, D), c)],
        out_shape=[jax.ShapeDtypeStruct((E, D), F32),
                   jax.ShapeDtypeStruct((E, 8), F32),
                   jax.ShapeDtypeStruct((E, D), F32)],
    )(f, hs, hd, Wni, Wnj, Wfij, Wnode, attnM)


def _tc_edge_l2(f, hs, hd, Wni, Wnj, Wfij, Wnode, attnM):
    def body(f_ref, hs0_ref, hs1_ref, hs2_ref, hs3_ref, hd0_ref, hd1_ref,
             hd2_ref, hd3_ref, wi_ref, wj_ref, wf_ref, wn_ref, am_ref,
             fo_ref, ex_ref, hn_ref):
        wi = wi_ref[...]
        wj = wj_ref[...]
        wn = wn_ref[...]
        hs_r = (hs0_ref, hs1_ref, hs2_ref, hs3_ref)
        hd_r = (hd0_ref, hd1_ref, hd2_ref, hd3_ref)
        fo = jnp.dot(f_ref[...], wf_ref[...], preferred_element_type=F32)
        hn = jnp.zeros((BE, D), F32)
        for q in range(4):
            sl = slice(q * HD, (q + 1) * HD)
            hsq = hs_r[q][...]
            fo = fo + jnp.dot(hsq, wi[sl], preferred_element_type=F32) \
                + jnp.dot(hd_r[q][...], wj[sl], preferred_element_type=F32)
            hn = hn + jnp.dot(hsq, wn[sl], preferred_element_type=F32)
        fo_ref[...] = fo
        lr = jnp.where(fo >= 0.0, fo, 0.2 * fo)
        l8 = jnp.dot(lr, am_ref[...], preferred_element_type=F32)
        ex_ref[...] = jnp.exp(jnp.clip(l8, -80.0, 80.0))
        hn_ref[...] = hn

    c = lambda i: (i, 0)
    z = lambda i: (0, 0)
    q_i = pl.BlockSpec((BE, HD), c)
    return pl.pallas_call(
        body,
        grid=(E // BE,),
        in_specs=[pl.BlockSpec((BE, D), c)] + [q_i] * 8 + [
            pl.BlockSpec((D, D), z), pl.BlockSpec((D, D), z),
            pl.BlockSpec((D, D), z), pl.BlockSpec((D, D), z),
            pl.BlockSpec((D, 8), z),
        ],
        out_specs=[pl.BlockSpec((BE, D), c), pl.BlockSpec((BE, 8), c),
                   pl.BlockSpec((BE, D), c)],
        out_shape=[jax.ShapeDtypeStruct((E, D), F32),
                   jax.ShapeDtypeStruct((E, 8), F32),
                   jax.ShapeDtypeStruct((E, D), F32)],
    )(f, *hs, *hd, Wni, Wnj, Wfij, Wnode, attnM)


def _msg(hn_src, expl, s1_dst):
    def body(hn_ref, ex_ref, s1_ref, m0_ref, m1_ref, m2_ref, m3_ref):
        a8 = ex_ref[...] / s1_ref[...]
        hn = hn_ref[...]
        outs = (m0_ref, m1_ref, m2_ref, m3_ref)
        for q in range(4):
            aq = jnp.broadcast_to(a8[:, q:q + 1], (BE, HD))
            outs[q][...] = hn[:, q * HD:(q + 1) * HD] * aq

    q_o = pl.BlockSpec((BE, HD), lambda i: (i, 0))
    return pl.pallas_call(
        body,
        grid=(E // BE,),
        in_specs=[
            pl.BlockSpec((BE, D), lambda i: (i, 0)),
            pl.BlockSpec((BE, 8), lambda i: (i, 0)),
            pl.BlockSpec((BE, 8), lambda i: (i, 0)),
        ],
        out_specs=[q_o, q_o, q_o, q_o],
        out_shape=[jax.ShapeDtypeStruct((E, HD), F32)] * 4,
    )(hn_src, expl, s1_dst)


def _pool4(feats, Ws, b2d, ids3, nblk):
    M = feats[0].shape[0]
    bm = M // nblk

    def body(x0_ref, x1_ref, x2_ref, x3_ref, w0_ref, w1_ref, w2_ref, w3_ref,
             b_ref, id_ref, out_ref):
        i = pl.program_id(0)
        x_r = (x0_ref, x1_ref, x2_ref, x3_ref)
        w_r = (w0_ref, w1_ref, w2_ref, w3_ref)
        ft = b_ref[...] + jnp.zeros((bm, HD), F32)
        for q in range(4):
            ft = ft + jnp.dot(x_r[q][...], w_r[q][...],
                              preferred_element_type=F32)
        ids = jnp.reshape(id_ref[0, 0, :], (bm, 1))
        oh = (ids == lax.broadcasted_iota(jnp.int32, (bm, G), 1)).astype(F32)
        part = lax.dot_general(oh, ft, (((0,), (0,)), ((), ())),
                               preferred_element_type=F32)

        @pl.when(i == 0)
        def _():
            out_ref[...] = jnp.zeros_like(out_ref)

        out_ref[...] += part

    return pl.pallas_call(
        body,
        grid=(nblk,),
        in_specs=[pl.BlockSpec((bm, HD), lambda i: (i, 0))] * 4
        + [pl.BlockSpec((HD, HD), lambda i: (0, 0))] * 4
        + [pl.BlockSpec((1, HD), lambda i: (0, 0)),
           pl.BlockSpec((1, 1, bm), lambda i: (i, 0, 0))],
        out_specs=pl.BlockSpec((G, HD), lambda i: (0, 0)),
        out_shape=jax.ShapeDtypeStruct((G, HD), F32),
    )(*feats, *Ws, b2d, ids3)


def _pool1(feat, W, b2d, ids3, nblk):
    M, din = feat.shape
    bm = M // nblk

    def body(x_ref, w_ref, b_ref, id_ref, out_ref):
        i = pl.program_id(0)
        ft = jnp.dot(x_ref[...], w_ref[...], preferred_element_type=F32) \
            + b_ref[...]
        ids = jnp.reshape(id_ref[0, 0, :], (bm, 1))
        oh = (ids == lax.broadcasted_iota(jnp.int32, (bm, G), 1)).astype(F32)
        part = lax.dot_general(oh, ft, (((0,), (0,)), ((), ())),
                               preferred_element_type=F32)

        @pl.when(i == 0)
        def _():
            out_ref[...] = jnp.zeros_like(out_ref)

        out_ref[...] += part

    return pl.pallas_call(
        body,
        grid=(nblk,),
        in_specs=[
            pl.BlockSpec((bm, din), lambda i: (i, 0)),
            pl.BlockSpec((din, HD), lambda i: (0, 0)),
            pl.BlockSpec((1, HD), lambda i: (0, 0)),
            pl.BlockSpec((1, 1, bm), lambda i: (i, 0, 0)),
        ],
        out_specs=pl.BlockSpec((G, HD), lambda i: (0, 0)),
        out_shape=jax.ShapeDtypeStruct((G, HD), F32),
    )(feat, W, b2d, ids3)


def _mlp(gn, ge, W1, b1, W2, b2, W3p, b3p):
    def body(gn_ref, ge_ref, w1_ref, b1_ref, w2_ref, b2_ref, w3_ref, b3_ref,
             out_ref):
        Gm = jnp.concatenate([gn_ref[...], ge_ref[...]], axis=1)
        z = jnp.maximum(jnp.dot(Gm, w1_ref[...], preferred_element_type=F32)
                        + b1_ref[...], 0.0)
        z = jnp.maximum(jnp.dot(z, w2_ref[...], preferred_element_type=F32)
                        + b2_ref[...], 0.0)
        out_ref[...] = jnp.dot(z, w3_ref[...], preferred_element_type=F32) \
            + b3_ref[...]

    return pl.pallas_call(
        body,
        out_shape=jax.ShapeDtypeStruct((G, 8), F32),
    )(gn, ge, W1, b1, W2, b2, W3p, b3p)


def _attn_mat(attn):
    cols = []
    for h_ in range(H):
        cols.append(jnp.zeros((D,), F32).at[h_ * HD:(h_ + 1) * HD].set(attn[h_]))
    cols += [jnp.zeros((D,), F32)] * 4
    return jnp.stack(cols, axis=1)


def _head_expand_mats():
    pa = jnp.zeros((8, 32), F32)
    pb = jnp.zeros((8, 32), F32)
    for h_ in range(2):
        pa = pa.at[h_, h_ * HD:(h_ + 1) * HD].set(1.0)
        pb = pb.at[2 + h_, h_ * HD:(h_ + 1) * HD].set(1.0)
    return pa, pb


def kernel(x, edge_feats, edge_src, edge_dst, node_graph_ids, edge_graph_ids,
           W_ni1, W_nj1, W_fij1, W_node1, attn1, W_ni2, W_nj2, W_fij2, W_node2,
           attn2, W_aggN, b_aggN, W_aggE, b_aggE, W_m1, b_m1, W_m2, b_m2, W_m3,
           b_m3):
    z8 = jnp.zeros((N, 8), F32)
    z16 = jnp.zeros((N, HD), F32)
    am1 = _attn_mat(attn1)
    am2 = _attn_mat(attn2)
    ngid3 = jnp.reshape(node_graph_ids, (N // BN, 1, BN))
    egid3 = jnp.reshape(edge_graph_ids, (E // BE, 1, BE))
    src2d = jnp.reshape(edge_src, (NEB, EB))
    dst2d = jnp.reshape(edge_dst, (NEB, EB))
    x32 = jnp.pad(x, ((0, 0), (0, 32 - x.shape[1])))
    Wni1p = jnp.pad(W_ni1, ((0, 32 - W_ni1.shape[0]), (0, 0)))
    Wnj1p = jnp.pad(W_nj1, ((0, 32 - W_nj1.shape[0]), (0, 0)))
    Wnode1p = jnp.pad(W_node1, ((0, 32 - W_node1.shape[0]), (0, 0)))

    f = edge_feats
    hq = None
    for layer in range(3):
        if layer == 0:
            hs, hd = _sc_gather([x32], [src2d, dst2d], SB=5)
            fo, ex, hns = _tc_edge_l1(f, hs, hd, Wni1p, Wnj1p, W_fij1,
                                      Wnode1p, am1)
        else:
            g8 = _sc_gather(hq, [src2d, dst2d], SB=2)
            fo, ex, hns = _tc_edge_l2(f, g8[0::2], g8[1::2], W_ni2, W_nj2,
                                      W_fij2, W_node2, am2)
        f = fo
        s1d = _sc_s1(ex, dst2d, z8)
        msgs = _msg(hns, ex, s1d)
        hq = _sc_scatter_msg(msgs, dst2d, z16)

    wn_q = [W_aggN[q * HD:(q + 1) * HD] for q in range(4)]
    gn = _pool4(hq, wn_q, jnp.reshape(b_aggN, (1, HD)), ngid3, N // BN)
    ge = _pool1(f, W_aggE, jnp.reshape(b_aggE, (1, HD)), egid3, E // BE)
    W3p = jnp.zeros((HD, 8), F32).at[:, :1].set(W_m3)
    b3p = jnp.zeros((1, 8), F32).at[:, :1].set(jnp.reshape(b_m3, (1, 1)))
    out = _mlp(gn, ge, W_m1, jnp.reshape(b_m1, (1, HD)), W_m2,
               jnp.reshape(b_m2, (1, HD)), W3p, b3p)
    return out[:, :1]


# speedup vs baseline: 1.2483x; 1.2483x over previous
import functools

import jax
import jax.numpy as jnp
from jax import lax
from jax.experimental import pallas as pl
from jax.experimental.pallas import tpu as pltpu
from jax.experimental.pallas import tpu_sc as plsc

N = 50000
E = 800000
D = 64
H = 4
HD = 16
G = 512
EB = 128
NEB = E // EB
NC, NS = 2, 16
NW = NC * NS
NROW = N // NS
BE = 2000
BN = 2000
F32 = jnp.float32

_SC_PARAMS = dict(
    compiler_params=pltpu.CompilerParams(use_tc_tiling_on_sc=False),
)


def _mesh():
    return plsc.VectorSubcoreMesh(core_axis_name="c", subcore_axis_name="s")


def _sc_gather(tables, idx2ds, SB):
    NT = len(tables)
    NI = len(idx2ds)
    dds = [int(t.shape[1]) for t in tables]
    NSB = NEB // SB
    SBR = SB * EB

    outs = [jax.ShapeDtypeStruct((E, dds[tt]), F32)
            for tt in range(NT) for _ in range(NI)]
    scratch = []
    for b in range(2):
        for ii in range(NI):
            scratch.append(pltpu.VMEM((SB, EB), jnp.int32))
    for b in range(2):
        for tt in range(NT):
            for ii in range(NI):
                scratch.append(pltpu.VMEM((SBR, dds[tt]), F32))
    scratch += [pltpu.SemaphoreType.DMA] * 4

    @functools.partial(
        pl.kernel, out_type=outs, mesh=_mesh(),
        scratch_types=scratch, **_SC_PARAMS)
    def k(*refs):
        tab_h = refs[:NT]
        idx_h = refs[NT:NT + NI]
        out_h = refs[NT + NI:NT + NI + NT * NI]
        p = NT + NI + NT * NI
        iv = [[refs[p + b * NI + ii] for ii in range(NI)] for b in range(2)]
        p += 2 * NI
        buf = [[refs[p + b * NT * NI + q] for q in range(NT * NI)]
               for b in range(2)]
        p += 2 * NT * NI
        semG = [refs[p], refs[p + 1]]
        semW = [refs[p + 2], refs[p + 3]]

        wid = lax.axis_index("s") * NC + lax.axis_index("c")
        T = (NSB - wid + NW - 1) // NW

        def wait_wb(b):
            for q in range(NT * NI):
                pltpu.make_async_copy(
                    buf[b][q], out_h[q].at[pl.ds(0, SBR)], semW[b]).wait()

        def body(i, carry):
            for b in range(2):
                t = 2 * i + b

                @pl.when(t < T)
                def _(b=b, t=t):
                    sb = wid + t * NW
                    off = pl.multiple_of(sb * SBR, EB)

                    @pl.when(t >= 2)
                    def _():
                        wait_wb(b)

                    for ii in range(NI):
                        pltpu.sync_copy(idx_h[ii].at[pl.ds(sb * SB, SB)],
                                        iv[b][ii])
                    descs = []
                    for tt in range(NT):
                        for ii in range(NI):
                            for j in range(SB):
                                descs.append(pltpu.async_copy(
                                    tab_h[tt].at[iv[b][ii].at[j]],
                                    buf[b][tt * NI + ii].at[pl.ds(j * EB, EB)],
                                    semG[b]))
                    for dsc in descs:
                        dsc.wait()
                    for q in range(NT * NI):
                        pltpu.async_copy(buf[b][q], out_h[q].at[pl.ds(off, SBR)],
                                         semW[b])
            return carry

        lax.fori_loop(0, (T + 1) // 2, body, 0)
        wait_wb(0)
        wait_wb(1)

    return k(*tables, *idx2ds)


def _sc_s1(expl, dst2d, z8):
    SB = 5
    SBR = SB * EB
    NSB = NEB // SB

    @functools.partial(
        pl.kernel, out_type=jax.ShapeDtypeStruct((E, 8), F32), mesh=_mesh(),
        scratch_types=[
            pltpu.VMEM((SB, EB), jnp.int32), pltpu.VMEM((SB, EB), jnp.int32),
            pltpu.VMEM((SBR, 8), F32), pltpu.VMEM((SBR, 8), F32),
            pltpu.VMEM((SB, EB), jnp.int32), pltpu.VMEM((SB, EB), jnp.int32),
            pltpu.VMEM((SBR, 8), F32), pltpu.VMEM((SBR, 8), F32),
            pltpu.VMEM_SHARED((N, 8), F32),
            pltpu.SemaphoreType.DMA, pltpu.SemaphoreType.DMA,
            pltpu.SemaphoreType.DMA, pltpu.SemaphoreType.DMA,
            pltpu.SemaphoreType.DMA, pltpu.SemaphoreType.DMA,
        ], **_SC_PARAMS)
    def k(ex_h, di_h, z_h, s1d_h, ivS0, ivS1, bufS0, bufS1, ivG0, ivG1,
          bufG0, bufG1, acc, sS0, sS1, sG0, sG1, sW0, sW1):
        ivS = [ivS0, ivS1]
        bufS = [bufS0, bufS1]
        ivG = [ivG0, ivG1]
        bufG = [bufG0, bufG1]
        semS = [sS0, sS1]
        semG = [sG0, sG1]
        semW = [sW0, sW1]
        cid = lax.axis_index("c")
        sid = lax.axis_index("s")
        wid = sid * NC + cid
        r0 = pl.multiple_of(sid * NROW, NROW)
        pltpu.sync_copy(z_h.at[pl.ds(r0, NROW)], acc.at[pl.ds(r0, NROW)])
        plsc.subcore_barrier()

        Ts = (NSB - sid + NS - 1) // NS

        def wait_sc(b):
            for j in range(SB):
                pltpu.make_async_copy(bufS[b].at[pl.ds(j * EB, EB)],
                                      acc.at[ivS[b].at[j]], semS[b]).wait()

        def s_body(i, carry):
            for b in range(2):
                t = 2 * i + b

                @pl.when(t < Ts)
                def _(b=b, t=t):
                    sb = sid + t * NS
                    off = pl.multiple_of(sb * SBR, EB)

                    @pl.when(t >= 2)
                    def _():
                        wait_sc(b)

                    pltpu.sync_copy(di_h.at[pl.ds(sb * SB, SB)], ivS[b])
                    pltpu.sync_copy(ex_h.at[pl.ds(off, SBR)], bufS[b])
                    for j in range(SB):
                        pltpu.async_copy(bufS[b].at[pl.ds(j * EB, EB)],
                                         acc.at[ivS[b].at[j]], semS[b],
                                         add=True)
            return carry

        lax.fori_loop(0, (Ts + 1) // 2, s_body, 0)
        wait_sc(0)
        wait_sc(1)
        plsc.subcore_barrier()

        Tg = (NSB - wid + NW - 1) // NW

        def wait_wb(b):
            pltpu.make_async_copy(bufG[b], s1d_h.at[pl.ds(0, SBR)],
                                  semW[b]).wait()

        def g_body(i, carry):
            for b in range(2):
                t = 2 * i + b

                @pl.when(t < Tg)
                def _(b=b, t=t):
                    sb = wid + t * NW
                    off = pl.multiple_of(sb * SBR, EB)

                    @pl.when(t >= 2)
                    def _():
                        wait_wb(b)

                    pltpu.sync_copy(di_h.at[pl.ds(sb * SB, SB)], ivG[b])
                    descs = []
                    for j in range(SB):
                        descs.append(pltpu.async_copy(
                            acc.at[ivG[b].at[j]],
                            bufG[b].at[pl.ds(j * EB, EB)], semG[b]))
                    for dsc in descs:
                        dsc.wait()
                    pltpu.async_copy(bufG[b], s1d_h.at[pl.ds(off, SBR)],
                                     semW[b])
            return carry

        lax.fori_loop(0, (Tg + 1) // 2, g_body, 0)
        wait_wb(0)
        wait_wb(1)

    return k(expl, dst2d, z8)


def _sc_scatter_msg(msgA, msgB, dst2d, z32):
    SB = 2
    SBR = SB * EB
    NSB = NEB // SB

    @functools.partial(
        pl.kernel, out_type=[jax.ShapeDtypeStruct((N, 32), F32)] * 2,
        mesh=_mesh(),
        scratch_types=[
            pltpu.VMEM((SB, EB), jnp.int32), pltpu.VMEM((SB, EB), jnp.int32),
            pltpu.VMEM((SBR, 32), F32), pltpu.VMEM((SBR, 32), F32),
            pltpu.VMEM_SHARED((N, 32), F32),
            pltpu.SemaphoreType.DMA, pltpu.SemaphoreType.DMA,
        ], **_SC_PARAMS)
    def k(ma_h, mb_h, di_h, z_h, ha_h, hb_h, iv0, iv1, buf0, buf1, acc,
          sS0, sS1):
        iv = [iv0, iv1]
        buf = [buf0, buf1]
        semS = [sS0, sS1]
        cid = lax.axis_index("c")
        sid = lax.axis_index("s")
        r0 = pl.multiple_of(sid * NROW, NROW)
        Ts = (NSB - sid + NS - 1) // NS
        pltpu.sync_copy(z_h.at[pl.ds(r0, NROW)], acc.at[pl.ds(r0, NROW)])
        plsc.subcore_barrier()

        def wait_sc(b):
            for j in range(SB):
                pltpu.make_async_copy(buf[b].at[pl.ds(j * EB, EB)],
                                      acc.at[iv[b].at[j]], semS[b]).wait()

        def s_body(i, carry):
            for b in range(2):
                t = 2 * i + b

                @pl.when(t < Ts)
                def _(b=b, t=t):
                    sb = sid + t * NS
                    off = pl.multiple_of(sb * SBR, EB)

                    @pl.when(t >= 2)
                    def _():
                        wait_sc(b)

                    pltpu.sync_copy(di_h.at[pl.ds(sb * SB, SB)], iv[b])

                    @pl.when(cid == 0)
                    def _():
                        pltpu.sync_copy(ma_h.at[pl.ds(off, SBR)], buf[b])

                    @pl.when(cid == 1)
                    def _():
                        pltpu.sync_copy(mb_h.at[pl.ds(off, SBR)], buf[b])

                    for j in range(SB):
                        pltpu.async_copy(buf[b].at[pl.ds(j * EB, EB)],
                                         acc.at[iv[b].at[j]], semS[b],
                                         add=True)
            return carry

        lax.fori_loop(0, (Ts + 1) // 2, s_body, 0)
        wait_sc(0)
        wait_sc(1)
        plsc.subcore_barrier()

        @pl.when(cid == 0)
        def _():
            pltpu.sync_copy(acc.at[pl.ds(r0, NROW)], ha_h.at[pl.ds(r0, NROW)])

        @pl.when(cid == 1)
        def _():
            pltpu.sync_copy(acc.at[pl.ds(r0, NROW)], hb_h.at[pl.ds(r0, NROW)])

    return k(msgA, msgB, dst2d, z32)


def _tc_edge_l1(f, hs, hd, Wni, Wnj, Wfij, Wnode, attnM):
    fin = f.shape[1]

    def body(f_ref, hs_ref, hd_ref, wi_ref, wj_ref, wf_ref, wn_ref, am_ref,
             fo_ref, ex_ref, hn_ref):
        hs = hs_ref[...]
        fo = jnp.dot(hs, wi_ref[...], preferred_element_type=F32) \
            + jnp.dot(hd_ref[...], wj_ref[...], preferred_element_type=F32) \
            + jnp.dot(f_ref[...], wf_ref[...], preferred_element_type=F32)
        fo_ref[...] = fo
        lr = jnp.where(fo >= 0.0, fo, 0.2 * fo)
        l8 = jnp.dot(lr, am_ref[...], preferred_element_type=F32,
                     precision=lax.Precision.HIGHEST)
        ex_ref[...] = jnp.exp(jnp.clip(l8, -80.0, 80.0))
        hn_ref[...] = jnp.dot(hs, wn_ref[...], preferred_element_type=F32)

    c = lambda i: (i, 0)
    z = lambda i: (0, 0)
    return pl.pallas_call(
        body,
        grid=(E // BE,),
        in_specs=[
            pl.BlockSpec((BE, fin), c), pl.BlockSpec((BE, 32), c),
            pl.BlockSpec((BE, 32), c), pl.BlockSpec((32, D), z),
            pl.BlockSpec((32, D), z), pl.BlockSpec((fin, D), z),
            pl.BlockSpec((32, D), z), pl.BlockSpec((D, 8), z),
        ],
        out_specs=[pl.BlockSpec((BE, D), c), pl.BlockSpec((BE, 8), c),
                   pl.BlockSpec((BE, D), c)],
        out_shape=[jax.ShapeDtypeStruct((E, D), F32),
                   jax.ShapeDtypeStruct((E, 8), F32),
                   jax.ShapeDtypeStruct((E, D), F32)],
    )(f, hs, hd, Wni, Wnj, Wfij, Wnode, attnM)


def _tc_edge_l2(f, hsA, hsB, hdA, hdB, Wni, Wnj, Wfij, Wnode, attnM):
    def body(f_ref, hsA_ref, hsB_ref, hdA_ref, hdB_ref, wi_ref, wj_ref,
             wf_ref, wn_ref, am_ref, fo_ref, ex_ref, hn_ref):
        hs = jnp.concatenate([hsA_ref[...], hsB_ref[...]], axis=1)
        hd = jnp.concatenate([hdA_ref[...], hdB_ref[...]], axis=1)
        fo = jnp.dot(hs, wi_ref[...], preferred_element_type=F32) \
            + jnp.dot(hd, wj_ref[...], preferred_element_type=F32) \
            + jnp.dot(f_ref[...], wf_ref[...], preferred_element_type=F32)
        fo_ref[...] = fo
        lr = jnp.where(fo >= 0.0, fo, 0.2 * fo)
        l8 = jnp.dot(lr, am_ref[...], preferred_element_type=F32,
                     precision=lax.Precision.HIGHEST)
        ex_ref[...] = jnp.exp(jnp.clip(l8, -80.0, 80.0))
        hn_ref[...] = jnp.dot(hs, wn_ref[...], preferred_element_type=F32)

    c = lambda i: (i, 0)
    z = lambda i: (0, 0)
    h_i = pl.BlockSpec((BE, 32), c)
    return pl.pallas_call(
        body,
        grid=(E // BE,),
        in_specs=[pl.BlockSpec((BE, D), c), h_i, h_i, h_i, h_i,
                  pl.BlockSpec((D, D), z), pl.BlockSpec((D, D), z),
                  pl.BlockSpec((D, D), z), pl.BlockSpec((D, D), z),
                  pl.BlockSpec((D, 8), z)],
        out_specs=[pl.BlockSpec((BE, D), c), pl.BlockSpec((BE, 8), c),
                   pl.BlockSpec((BE, D), c)],
        out_shape=[jax.ShapeDtypeStruct((E, D), F32),
                   jax.ShapeDtypeStruct((E, 8), F32),
                   jax.ShapeDtypeStruct((E, D), F32)],
    )(f, hsA, hsB, hdA, hdB, Wni, Wnj, Wfij, Wnode, attnM)


def _msg(hn_src, expl, s1_dst, PA, PB):
    def body(hn_ref, ex_ref, s1_ref, pa_ref, pb_ref, ma_ref, mb_ref):
        a8 = ex_ref[...] / s1_ref[...]
        arepA = jnp.dot(a8, pa_ref[...], preferred_element_type=F32)
        arepB = jnp.dot(a8, pb_ref[...], preferred_element_type=F32)
        hn = hn_ref[...]
        ma_ref[...] = hn[:, :32] * arepA
        mb_ref[...] = hn[:, 32:] * arepB

    half_o = pl.BlockSpec((BE, 32), lambda i: (i, 0))
    return pl.pallas_call(
        body,
        grid=(E // BE,),
        in_specs=[
            pl.BlockSpec((BE, D), lambda i: (i, 0)),
            pl.BlockSpec((BE, 8), lambda i: (i, 0)),
            pl.BlockSpec((BE, 8), lambda i: (i, 0)),
            pl.BlockSpec((8, 32), lambda i: (0, 0)),
            pl.BlockSpec((8, 32), lambda i: (0, 0)),
        ],
        out_specs=[half_o, half_o],
        out_shape=[jax.ShapeDtypeStruct((E, 32), F32)] * 2,
    )(hn_src, expl, s1_dst, PA, PB)


def _pool2(featA, featB, WT, WB, b2d, ids3, nblk):
    M = featA.shape[0]
    bm = M // nblk

    def body(xa_ref, xb_ref, wt_ref, wb_ref, b_ref, id_ref, out_ref):
        i = pl.program_id(0)
        ft = jnp.dot(xa_ref[...], wt_ref[...], preferred_element_type=F32) \
            + jnp.dot(xb_ref[...], wb_ref[...], preferred_element_type=F32) \
            + b_ref[...]
        ids = jnp.reshape(id_ref[0, 0, :], (bm, 1))
        oh = (ids == lax.broadcasted_iota(jnp.int32, (bm, G), 1)).astype(F32)
        part = lax.dot_general(oh, ft, (((0,), (0,)), ((), ())),
                               preferred_element_type=F32,
                               precision=lax.Precision.HIGHEST)

        @pl.when(i == 0)
        def _():
            out_ref[...] = jnp.zeros_like(out_ref)

        out_ref[...] += part

    return pl.pallas_call(
        body,
        grid=(nblk,),
        in_specs=[
            pl.BlockSpec((bm, 32), lambda i: (i, 0)),
            pl.BlockSpec((bm, 32), lambda i: (i, 0)),
            pl.BlockSpec((32, HD), lambda i: (0, 0)),
            pl.BlockSpec((32, HD), lambda i: (0, 0)),
            pl.BlockSpec((1, HD), lambda i: (0, 0)),
            pl.BlockSpec((1, 1, bm), lambda i: (i, 0, 0)),
        ],
        out_specs=pl.BlockSpec((G, HD), lambda i: (0, 0)),
        out_shape=jax.ShapeDtypeStruct((G, HD), F32),
    )(featA, featB, WT, WB, b2d, ids3)


def _pool1(feat, W, b2d, ids3, nblk):
    M, din = feat.shape
    bm = M // nblk

    def body(x_ref, w_ref, b_ref, id_ref, out_ref):
        i = pl.program_id(0)
        ft = jnp.dot(x_ref[...], w_ref[...], preferred_element_type=F32) \
            + b_ref[...]
        ids = jnp.reshape(id_ref[0, 0, :], (bm, 1))
        oh = (ids == lax.broadcasted_iota(jnp.int32, (bm, G), 1)).astype(F32)
        part = lax.dot_general(oh, ft, (((0,), (0,)), ((), ())),
                               preferred_element_type=F32,
                               precision=lax.Precision.HIGHEST)

        @pl.when(i == 0)
        def _():
            out_ref[...] = jnp.zeros_like(out_ref)

        out_ref[...] += part

    return pl.pallas_call(
        body,
        grid=(nblk,),
        in_specs=[
            pl.BlockSpec((bm, din), lambda i: (i, 0)),
            pl.BlockSpec((din, HD), lambda i: (0, 0)),
            pl.BlockSpec((1, HD), lambda i: (0, 0)),
            pl.BlockSpec((1, 1, bm), lambda i: (i, 0, 0)),
        ],
        out_specs=pl.BlockSpec((G, HD), lambda i: (0, 0)),
        out_shape=jax.ShapeDtypeStruct((G, HD), F32),
    )(feat, W, b2d, ids3)


def _mlp(gn, ge, W1, b1, W2, b2, W3p, b3p):
    def body(gn_ref, ge_ref, w1_ref, b1_ref, w2_ref, b2_ref, w3_ref, b3_ref,
             out_ref):
        Gm = jnp.concatenate([gn_ref[...], ge_ref[...]], axis=1)
        z = jnp.maximum(jnp.dot(Gm, w1_ref[...], preferred_element_type=F32)
                        + b1_ref[...], 0.0)
        z = jnp.maximum(jnp.dot(z, w2_ref[...], preferred_element_type=F32)
                        + b2_ref[...], 0.0)
        out_ref[...] = jnp.dot(z, w3_ref[...], preferred_element_type=F32) \
            + b3_ref[...]

    return pl.pallas_call(
        body,
        out_shape=jax.ShapeDtypeStruct((G, 8), F32),
    )(gn, ge, W1, b1, W2, b2, W3p, b3p)


def _attn_mat(attn):
    cols = []
    for h_ in range(H):
        cols.append(jnp.zeros((D,), F32).at[h_ * HD:(h_ + 1) * HD].set(attn[h_]))
    cols += [jnp.zeros((D,), F32)] * 4
    return jnp.stack(cols, axis=1)


def _head_expand_mats():
    pa = jnp.zeros((8, 32), F32)
    pb = jnp.zeros((8, 32), F32)
    for h_ in range(2):
        pa = pa.at[h_, h_ * HD:(h_ + 1) * HD].set(1.0)
        pb = pb.at[2 + h_, h_ * HD:(h_ + 1) * HD].set(1.0)
    return pa, pb


def kernel(x, edge_feats, edge_src, edge_dst, node_graph_ids, edge_graph_ids,
           W_ni1, W_nj1, W_fij1, W_node1, attn1, W_ni2, W_nj2, W_fij2, W_node2,
           attn2, W_aggN, b_aggN, W_aggE, b_aggE, W_m1, b_m1, W_m2, b_m2, W_m3,
           b_m3):
    z8 = jnp.zeros((N, 8), F32)
    z32 = jnp.zeros((N, 32), F32)
    am1 = _attn_mat(attn1)
    am2 = _attn_mat(attn2)
    PA, PB = _head_expand_mats()
    ngid3 = jnp.reshape(node_graph_ids, (N // BN, 1, BN))
    egid3 = jnp.reshape(edge_graph_ids, (E // BE, 1, BE))
    src2d = jnp.reshape(edge_src, (NEB, EB))
    dst2d = jnp.reshape(edge_dst, (NEB, EB))
    x32 = jnp.pad(x, ((0, 0), (0, 32 - x.shape[1])))
    Wni1p = jnp.pad(W_ni1, ((0, 32 - W_ni1.shape[0]), (0, 0)))
    Wnj1p = jnp.pad(W_nj1, ((0, 32 - W_nj1.shape[0]), (0, 0)))
    Wnode1p = jnp.pad(W_node1, ((0, 32 - W_node1.shape[0]), (0, 0)))

    f = edge_feats
    ha = hb = None
    for layer in range(3):
        if layer == 0:
            hs, hd = _sc_gather([x32], [src2d, dst2d], SB=5)
            fo, ex, hns = _tc_edge_l1(f, hs, hd, Wni1p, Wnj1p, W_fij1,
                                      Wnode1p, am1)
        else:
            hsA, hdA, hsB, hdB = _sc_gather([ha, hb], [src2d, dst2d], SB=2)
            fo, ex, hns = _tc_edge_l2(f, hsA, hsB, hdA, hdB, W_ni2, W_nj2,
                                      W_fij2, W_node2, am2)
        f = fo
        s1d = _sc_s1(ex, dst2d, z8)
        ma, mb = _msg(hns, ex, s1d, PA, PB)
        ha, hb = _sc_scatter_msg(ma, mb, dst2d, z32)

    gn = _pool2(ha, hb, W_aggN[:32], W_aggN[32:], jnp.reshape(b_aggN, (1, HD)),
                ngid3, N // BN)
    ge = _pool1(f, W_aggE, jnp.reshape(b_aggE, (1, HD)), egid3, E // BE)
    W3p = jnp.zeros((HD, 8), F32).at[:, :1].set(W_m3)
    b3p = jnp.zeros((1, 8), F32).at[:, :1].set(jnp.reshape(b_m3, (1, 1)))
    out = _mlp(gn, ge, W_m1, jnp.reshape(b_m1, (1, HD)), W_m2,
               jnp.reshape(b_m2, (1, HD)), W3p, b3p)
    return out[:, :1]


# speedup vs baseline: 1.3070x; 1.0471x over previous
import functools

import jax
import jax.numpy as jnp
from jax import lax
from jax.experimental import pallas as pl
from jax.experimental.pallas import tpu as pltpu
from jax.experimental.pallas import tpu_sc as plsc

N = 50000
E = 800000
D = 64
H = 4
HD = 16
G = 512
EB = 128
NEB = E // EB
NC, NS = 2, 16
NW = NC * NS
NROW = N // NS
BE = 4000
BN = 2000
F32 = jnp.float32

_SC_PARAMS = dict(
    compiler_params=pltpu.CompilerParams(use_tc_tiling_on_sc=False),
)


def _mesh():
    return plsc.VectorSubcoreMesh(core_axis_name="c", subcore_axis_name="s")


def _sc_gather(tables, idx2ds, SB):
    NT = len(tables)
    NI = len(idx2ds)
    dds = [int(t.shape[1]) for t in tables]
    NSB = NEB // SB
    SBR = SB * EB

    outs = [jax.ShapeDtypeStruct((E, dds[tt]), F32)
            for tt in range(NT) for _ in range(NI)]
    scratch = []
    for b in range(2):
        for ii in range(NI):
            scratch.append(pltpu.VMEM((SB, EB), jnp.int32))
    for b in range(2):
        for tt in range(NT):
            for ii in range(NI):
                scratch.append(pltpu.VMEM((SBR, dds[tt]), F32))
    scratch += [pltpu.SemaphoreType.DMA] * 4

    @functools.partial(
        pl.kernel, out_type=outs, mesh=_mesh(),
        scratch_types=scratch, **_SC_PARAMS)
    def k(*refs):
        tab_h = refs[:NT]
        idx_h = refs[NT:NT + NI]
        out_h = refs[NT + NI:NT + NI + NT * NI]
        p = NT + NI + NT * NI
        iv = [[refs[p + b * NI + ii] for ii in range(NI)] for b in range(2)]
        p += 2 * NI
        buf = [[refs[p + b * NT * NI + q] for q in range(NT * NI)]
               for b in range(2)]
        p += 2 * NT * NI
        semG = [refs[p], refs[p + 1]]
        semW = [refs[p + 2], refs[p + 3]]

        wid = lax.axis_index("s") * NC + lax.axis_index("c")
        T = (NSB - wid + NW - 1) // NW

        def wait_wb(b):
            for q in range(NT * NI):
                pltpu.make_async_copy(
                    buf[b][q], out_h[q].at[pl.ds(0, SBR)], semW[b]).wait()

        def body(i, carry):
            for b in range(2):
                t = 2 * i + b

                @pl.when(t < T)
                def _(b=b, t=t):
                    sb = wid + t * NW
                    off = pl.multiple_of(sb * SBR, EB)

                    @pl.when(t >= 2)
                    def _():
                        wait_wb(b)

                    for ii in range(NI):
                        pltpu.sync_copy(idx_h[ii].at[pl.ds(sb * SB, SB)],
                                        iv[b][ii])
                    descs = []
                    for tt in range(NT):
                        for ii in range(NI):
                            for j in range(SB):
                                descs.append(pltpu.async_copy(
                                    tab_h[tt].at[iv[b][ii].at[j]],
                                    buf[b][tt * NI + ii].at[pl.ds(j * EB, EB)],
                                    semG[b]))
                    for dsc in descs:
                        dsc.wait()
                    for q in range(NT * NI):
                        pltpu.async_copy(buf[b][q], out_h[q].at[pl.ds(off, SBR)],
                                         semW[b])
            return carry

        lax.fori_loop(0, (T + 1) // 2, body, 0)
        wait_wb(0)
        wait_wb(1)

    return k(*tables, *idx2ds)


def _sc_s1(expl, dst2d, z8):
    SB = 5
    SBR = SB * EB
    NSB = NEB // SB

    @functools.partial(
        pl.kernel, out_type=jax.ShapeDtypeStruct((E, 8), F32), mesh=_mesh(),
        scratch_types=[
            pltpu.VMEM((SB, EB), jnp.int32), pltpu.VMEM((SB, EB), jnp.int32),
            pltpu.VMEM((SBR, 8), F32), pltpu.VMEM((SBR, 8), F32),
            pltpu.VMEM((SB, EB), jnp.int32), pltpu.VMEM((SB, EB), jnp.int32),
            pltpu.VMEM((SBR, 8), F32), pltpu.VMEM((SBR, 8), F32),
            pltpu.VMEM_SHARED((N, 8), F32),
            pltpu.SemaphoreType.DMA, pltpu.SemaphoreType.DMA,
            pltpu.SemaphoreType.DMA, pltpu.SemaphoreType.DMA,
            pltpu.SemaphoreType.DMA, pltpu.SemaphoreType.DMA,
        ], **_SC_PARAMS)
    def k(ex_h, di_h, z_h, s1d_h, ivS0, ivS1, bufS0, bufS1, ivG0, ivG1,
          bufG0, bufG1, acc, sS0, sS1, sG0, sG1, sW0, sW1):
        ivS = [ivS0, ivS1]
        bufS = [bufS0, bufS1]
        ivG = [ivG0, ivG1]
        bufG = [bufG0, bufG1]
        semS = [sS0, sS1]
        semG = [sG0, sG1]
        semW = [sW0, sW1]
        cid = lax.axis_index("c")
        sid = lax.axis_index("s")
        wid = sid * NC + cid
        r0 = pl.multiple_of(sid * NROW, NROW)
        pltpu.sync_copy(z_h.at[pl.ds(r0, NROW)], acc.at[pl.ds(r0, NROW)])
        plsc.subcore_barrier()

        Ts = (NSB - sid + NS - 1) // NS

        def wait_sc(b):
            for j in range(SB):
                pltpu.make_async_copy(bufS[b].at[pl.ds(j * EB, EB)],
                                      acc.at[ivS[b].at[j]], semS[b]).wait()

        def s_body(i, carry):
            for b in range(2):
                t = 2 * i + b

                @pl.when(t < Ts)
                def _(b=b, t=t):
                    sb = sid + t * NS
                    off = pl.multiple_of(sb * SBR, EB)

                    @pl.when(t >= 2)
                    def _():
                        wait_sc(b)

                    pltpu.sync_copy(di_h.at[pl.ds(sb * SB, SB)], ivS[b])
                    pltpu.sync_copy(ex_h.at[pl.ds(off, SBR)], bufS[b])
                    for j in range(SB):
                        pltpu.async_copy(bufS[b].at[pl.ds(j * EB, EB)],
                                         acc.at[ivS[b].at[j]], semS[b],
                                         add=True)
            return carry

        lax.fori_loop(0, (Ts + 1) // 2, s_body, 0)
        wait_sc(0)
        wait_sc(1)
        plsc.subcore_barrier()

        Tg = (NSB - wid + NW - 1) // NW

        def wait_wb(b):
            pltpu.make_async_copy(bufG[b], s1d_h.at[pl.ds(0, SBR)],
                                  semW[b]).wait()

        def g_body(i, carry):
            for b in range(2):
                t = 2 * i + b

                @pl.when(t < Tg)
                def _(b=b, t=t):
                    sb = wid + t * NW
                    off = pl.multiple_of(sb * SBR, EB)

                    @pl.when(t >= 2)
                    def _():
                        wait_wb(b)

                    pltpu.sync_copy(di_h.at[pl.ds(sb * SB, SB)], ivG[b])
                    descs = []
                    for j in range(SB):
                        descs.append(pltpu.async_copy(
                            acc.at[ivG[b].at[j]],
                            bufG[b].at[pl.ds(j * EB, EB)], semG[b]))
                    for dsc in descs:
                        dsc.wait()
                    pltpu.async_copy(bufG[b], s1d_h.at[pl.ds(off, SBR)],
                                     semW[b])
            return carry

        lax.fori_loop(0, (Tg + 1) // 2, g_body, 0)
        wait_wb(0)
        wait_wb(1)

    return k(expl, dst2d, z8)


def _sc_scatter_msg(msgA, msgB, dst2d, z32):
    SB = 2
    SBR = SB * EB
    NSB = NEB // SB

    @functools.partial(
        pl.kernel, out_type=[jax.ShapeDtypeStruct((N, 32), F32)] * 2,
        mesh=_mesh(),
        scratch_types=[
            pltpu.VMEM((SB, EB), jnp.int32), pltpu.VMEM((SB, EB), jnp.int32),
            pltpu.VMEM((SBR, 32), F32), pltpu.VMEM((SBR, 32), F32),
            pltpu.VMEM_SHARED((N, 32), F32),
            pltpu.SemaphoreType.DMA, pltpu.SemaphoreType.DMA,
        ], **_SC_PARAMS)
    def k(ma_h, mb_h, di_h, z_h, ha_h, hb_h, iv0, iv1, buf0, buf1, acc,
          sS0, sS1):
        iv = [iv0, iv1]
        buf = [buf0, buf1]
        semS = [sS0, sS1]
        cid = lax.axis_index("c")
        sid = lax.axis_index("s")
        r0 = pl.multiple_of(sid * NROW, NROW)
        Ts = (NSB - sid + NS - 1) // NS
        pltpu.sync_copy(z_h.at[pl.ds(r0, NROW)], acc.at[pl.ds(r0, NROW)])
        plsc.subcore_barrier()

        def wait_sc(b):
            for j in range(SB):
                pltpu.make_async_copy(buf[b].at[pl.ds(j * EB, EB)],
                                      acc.at[iv[b].at[j]], semS[b]).wait()

        def s_body(i, carry):
            for b in range(2):
                t = 2 * i + b

                @pl.when(t < Ts)
                def _(b=b, t=t):
                    sb = sid + t * NS
                    off = pl.multiple_of(sb * SBR, EB)

                    @pl.when(t >= 2)
                    def _():
                        wait_sc(b)

                    pltpu.sync_copy(di_h.at[pl.ds(sb * SB, SB)], iv[b])

                    @pl.when(cid == 0)
                    def _():
                        pltpu.sync_copy(ma_h.at[pl.ds(off, SBR)], buf[b])

                    @pl.when(cid == 1)
                    def _():
                        pltpu.sync_copy(mb_h.at[pl.ds(off, SBR)], buf[b])

                    for j in range(SB):
                        pltpu.async_copy(buf[b].at[pl.ds(j * EB, EB)],
                                         acc.at[iv[b].at[j]], semS[b],
                                         add=True)
            return carry

        lax.fori_loop(0, (Ts + 1) // 2, s_body, 0)
        wait_sc(0)
        wait_sc(1)
        plsc.subcore_barrier()

        @pl.when(cid == 0)
        def _():
            pltpu.sync_copy(acc.at[pl.ds(r0, NROW)], ha_h.at[pl.ds(r0, NROW)])

        @pl.when(cid == 1)
        def _():
            pltpu.sync_copy(acc.at[pl.ds(r0, NROW)], hb_h.at[pl.ds(r0, NROW)])

    return k(msgA, msgB, dst2d, z32)


def _tc_edge_l1(f, hs, hd, Wni, Wnj, Wfij, Wnode, attnM):
    fin = f.shape[1]

    def body(f_ref, hs_ref, hd_ref, wi_ref, wj_ref, wf_ref, wn_ref, am_ref,
             fo_ref, ex_ref, hn_ref):
        hs = hs_ref[...]
        fo = jnp.dot(hs, wi_ref[...], preferred_element_type=F32) \
            + jnp.dot(hd_ref[...], wj_ref[...], preferred_element_type=F32) \
            + jnp.dot(f_ref[...], wf_ref[...], preferred_element_type=F32)
        fo_ref[...] = fo
        lr = jnp.where(fo >= 0.0, fo, 0.2 * fo)
        l8 = jnp.dot(lr, am_ref[...], preferred_element_type=F32,
                     precision=lax.Precision.HIGHEST)
        ex_ref[...] = jnp.exp(jnp.clip(l8, -80.0, 80.0))
        hn_ref[...] = jnp.dot(hs, wn_ref[...], preferred_element_type=F32)

    c = lambda i: (i, 0)
    z = lambda i: (0, 0)
    return pl.pallas_call(
        body,
        grid=(E // BE,),
        in_specs=[
            pl.BlockSpec((BE, fin), c), pl.BlockSpec((BE, 32), c),
            pl.BlockSpec((BE, 32), c), pl.BlockSpec((32, D), z),
            pl.BlockSpec((32, D), z), pl.BlockSpec((fin, D), z),
            pl.BlockSpec((32, D), z), pl.BlockSpec((D, 8), z),
        ],
        out_specs=[pl.BlockSpec((BE, D), c), pl.BlockSpec((BE, 8), c),
                   pl.BlockSpec((BE, D), c)],
        out_shape=[jax.ShapeDtypeStruct((E, D), F32),
                   jax.ShapeDtypeStruct((E, 8), F32),
                   jax.ShapeDtypeStruct((E, D), F32)],
    )(f, hs, hd, Wni, Wnj, Wfij, Wnode, attnM)


def _tc_edge_l2(f, hsA, hsB, hdA, hdB, Wni, Wnj, Wfij, Wnode, attnM):
    def body(f_ref, hsA_ref, hsB_ref, hdA_ref, hdB_ref, wi_ref, wj_ref,
             wf_ref, wn_ref, am_ref, fo_ref, ex_ref, hn_ref):
        hs = jnp.concatenate([hsA_ref[...], hsB_ref[...]], axis=1)
        hd = jnp.concatenate([hdA_ref[...], hdB_ref[...]], axis=1)
        fo = jnp.dot(hs, wi_ref[...], preferred_element_type=F32) \
            + jnp.dot(hd, wj_ref[...], preferred_element_type=F32) \
            + jnp.dot(f_ref[...], wf_ref[...], preferred_element_type=F32)
        fo_ref[...] = fo
        lr = jnp.where(fo >= 0.0, fo, 0.2 * fo)
        l8 = jnp.dot(lr, am_ref[...], preferred_element_type=F32,
                     precision=lax.Precision.HIGHEST)
        ex_ref[...] = jnp.exp(jnp.clip(l8, -80.0, 80.0))
        hn_ref[...] = jnp.dot(hs, wn_ref[...], preferred_element_type=F32)

    c = lambda i: (i, 0)
    z = lambda i: (0, 0)
    h_i = pl.BlockSpec((BE, 32), c)
    return pl.pallas_call(
        body,
        grid=(E // BE,),
        in_specs=[pl.BlockSpec((BE, D), c), h_i, h_i, h_i, h_i,
                  pl.BlockSpec((D, D), z), pl.BlockSpec((D, D), z),
                  pl.BlockSpec((D, D), z), pl.BlockSpec((D, D), z),
                  pl.BlockSpec((D, 8), z)],
        out_specs=[pl.BlockSpec((BE, D), c), pl.BlockSpec((BE, 8), c),
                   pl.BlockSpec((BE, D), c)],
        out_shape=[jax.ShapeDtypeStruct((E, D), F32),
                   jax.ShapeDtypeStruct((E, 8), F32),
                   jax.ShapeDtypeStruct((E, D), F32)],
    )(f, hsA, hsB, hdA, hdB, Wni, Wnj, Wfij, Wnode, attnM)


def _msg(hn_src, expl, s1_dst, PA, PB):
    def body(hn_ref, ex_ref, s1_ref, pa_ref, pb_ref, ma_ref, mb_ref):
        a8 = ex_ref[...] / s1_ref[...]
        arepA = jnp.dot(a8, pa_ref[...], preferred_element_type=F32)
        arepB = jnp.dot(a8, pb_ref[...], preferred_element_type=F32)
        hn = hn_ref[...]
        ma_ref[...] = hn[:, :32] * arepA
        mb_ref[...] = hn[:, 32:] * arepB

    half_o = pl.BlockSpec((BE, 32), lambda i: (i, 0))
    return pl.pallas_call(
        body,
        grid=(E // BE,),
        in_specs=[
            pl.BlockSpec((BE, D), lambda i: (i, 0)),
            pl.BlockSpec((BE, 8), lambda i: (i, 0)),
            pl.BlockSpec((BE, 8), lambda i: (i, 0)),
            pl.BlockSpec((8, 32), lambda i: (0, 0)),
            pl.BlockSpec((8, 32), lambda i: (0, 0)),
        ],
        out_specs=[half_o, half_o],
        out_shape=[jax.ShapeDtypeStruct((E, 32), F32)] * 2,
    )(hn_src, expl, s1_dst, PA, PB)


def _pool2(featA, featB, WT, WB, b2d, ids3, nblk):
    M = featA.shape[0]
    bm = M // nblk

    def body(xa_ref, xb_ref, wt_ref, wb_ref, b_ref, id_ref, out_ref):
        i = pl.program_id(0)
        ft = jnp.dot(xa_ref[...], wt_ref[...], preferred_element_type=F32) \
            + jnp.dot(xb_ref[...], wb_ref[...], preferred_element_type=F32) \
            + b_ref[...]
        ids = jnp.reshape(id_ref[0, 0, :], (bm, 1))
        oh = (ids == lax.broadcasted_iota(jnp.int32, (bm, G), 1)).astype(F32)
        part = lax.dot_general(oh, ft, (((0,), (0,)), ((), ())),
                               preferred_element_type=F32,
                               precision=lax.Precision.HIGHEST)

        @pl.when(i == 0)
        def _():
            out_ref[...] = jnp.zeros_like(out_ref)

        out_ref[...] += part

    return pl.pallas_call(
        body,
        grid=(nblk,),
        in_specs=[
            pl.BlockSpec((bm, 32), lambda i: (i, 0)),
            pl.BlockSpec((bm, 32), lambda i: (i, 0)),
            pl.BlockSpec((32, HD), lambda i: (0, 0)),
            pl.BlockSpec((32, HD), lambda i: (0, 0)),
            pl.BlockSpec((1, HD), lambda i: (0, 0)),
            pl.BlockSpec((1, 1, bm), lambda i: (i, 0, 0)),
        ],
        out_specs=pl.BlockSpec((G, HD), lambda i: (0, 0)),
        out_shape=jax.ShapeDtypeStruct((G, HD), F32),
    )(featA, featB, WT, WB, b2d, ids3)


def _pool1(feat, W, b2d, ids3, nblk):
    M, din = feat.shape
    bm = M // nblk

    def body(x_ref, w_ref, b_ref, id_ref, out_ref):
        i = pl.program_id(0)
        ft = jnp.dot(x_ref[...], w_ref[...], preferred_element_type=F32) \
            + b_ref[...]
        ids = jnp.reshape(id_ref[0, 0, :], (bm, 1))
        oh = (ids == lax.broadcasted_iota(jnp.int32, (bm, G), 1)).astype(F32)
        part = lax.dot_general(oh, ft, (((0,), (0,)), ((), ())),
                               preferred_element_type=F32,
                               precision=lax.Precision.HIGHEST)

        @pl.when(i == 0)
        def _():
            out_ref[...] = jnp.zeros_like(out_ref)

        out_ref[...] += part

    return pl.pallas_call(
        body,
        grid=(nblk,),
        in_specs=[
            pl.BlockSpec((bm, din), lambda i: (i, 0)),
            pl.BlockSpec((din, HD), lambda i: (0, 0)),
            pl.BlockSpec((1, HD), lambda i: (0, 0)),
            pl.BlockSpec((1, 1, bm), lambda i: (i, 0, 0)),
        ],
        out_specs=pl.BlockSpec((G, HD), lambda i: (0, 0)),
        out_shape=jax.ShapeDtypeStruct((G, HD), F32),
    )(feat, W, b2d, ids3)


def _mlp(gn, ge, W1, b1, W2, b2, W3p, b3p):
    def body(gn_ref, ge_ref, w1_ref, b1_ref, w2_ref, b2_ref, w3_ref, b3_ref,
             out_ref):
        Gm = jnp.concatenate([gn_ref[...], ge_ref[...]], axis=1)
        z = jnp.maximum(jnp.dot(Gm, w1_ref[...], preferred_element_type=F32)
                        + b1_ref[...], 0.0)
        z = jnp.maximum(jnp.dot(z, w2_ref[...], preferred_element_type=F32)
                        + b2_ref[...], 0.0)
        out_ref[...] = jnp.dot(z, w3_ref[...], preferred_element_type=F32) \
            + b3_ref[...]

    return pl.pallas_call(
        body,
        out_shape=jax.ShapeDtypeStruct((G, 8), F32),
    )(gn, ge, W1, b1, W2, b2, W3p, b3p)


def _attn_mat(attn):
    cols = []
    for h_ in range(H):
        cols.append(jnp.zeros((D,), F32).at[h_ * HD:(h_ + 1) * HD].set(attn[h_]))
    cols += [jnp.zeros((D,), F32)] * 4
    return jnp.stack(cols, axis=1)


def _head_expand_mats():
    pa = jnp.zeros((8, 32), F32)
    pb = jnp.zeros((8, 32), F32)
    for h_ in range(2):
        pa = pa.at[h_, h_ * HD:(h_ + 1) * HD].set(1.0)
        pb = pb.at[2 + h_, h_ * HD:(h_ + 1) * HD].set(1.0)
    return pa, pb


def kernel(x, edge_feats, edge_src, edge_dst, node_graph_ids, edge_graph_ids,
           W_ni1, W_nj1, W_fij1, W_node1, attn1, W_ni2, W_nj2, W_fij2, W_node2,
           attn2, W_aggN, b_aggN, W_aggE, b_aggE, W_m1, b_m1, W_m2, b_m2, W_m3,
           b_m3):
    z8 = jnp.zeros((N, 8), F32)
    z32 = jnp.zeros((N, 32), F32)
    am1 = _attn_mat(attn1)
    am2 = _attn_mat(attn2)
    PA, PB = _head_expand_mats()
    ngid3 = jnp.reshape(node_graph_ids, (N // BN, 1, BN))
    egid3 = jnp.reshape(edge_graph_ids, (E // BE, 1, BE))
    src2d = jnp.reshape(edge_src, (NEB, EB))
    dst2d = jnp.reshape(edge_dst, (NEB, EB))
    x32 = jnp.pad(x, ((0, 0), (0, 32 - x.shape[1])))
    Wni1p = jnp.pad(W_ni1, ((0, 32 - W_ni1.shape[0]), (0, 0)))
    Wnj1p = jnp.pad(W_nj1, ((0, 32 - W_nj1.shape[0]), (0, 0)))
    Wnode1p = jnp.pad(W_node1, ((0, 32 - W_node1.shape[0]), (0, 0)))

    f = edge_feats
    ha = hb = None
    for layer in range(3):
        if layer == 0:
            hs, hd = _sc_gather([x32], [src2d, dst2d], SB=5)
            fo, ex, hns = _tc_edge_l1(f, hs, hd, Wni1p, Wnj1p, W_fij1,
                                      Wnode1p, am1)
        else:
            hsA, hdA, hsB, hdB = _sc_gather([ha, hb], [src2d, dst2d], SB=2)
            fo, ex, hns = _tc_edge_l2(f, hsA, hsB, hdA, hdB, W_ni2, W_nj2,
                                      W_fij2, W_node2, am2)
        f = fo
        s1d = _sc_s1(ex, dst2d, z8)
        ma, mb = _msg(hns, ex, s1d, PA, PB)
        ha, hb = _sc_scatter_msg(ma, mb, dst2d, z32)

    gn = _pool2(ha, hb, W_aggN[:32], W_aggN[32:], jnp.reshape(b_aggN, (1, HD)),
                ngid3, N // BN)
    ge = _pool1(f, W_aggE, jnp.reshape(b_aggE, (1, HD)), egid3, E // BE)
    W3p = jnp.zeros((HD, 8), F32).at[:, :1].set(W_m3)
    b3p = jnp.zeros((1, 8), F32).at[:, :1].set(jnp.reshape(b_m3, (1, 1)))
    out = _mlp(gn, ge, W_m1, jnp.reshape(b_m1, (1, HD)), W_m2,
               jnp.reshape(b_m2, (1, HD)), W3p, b3p)
    return out[:, :1]
